# speedup vs baseline: 1.1637x; 1.1637x over previous
import jax
import jax.numpy as jnp
from jax.experimental import pallas as pl
from jax.experimental.pallas import tpu as pltpu

EPS = 1e-6
THETA = 1000000.0
H = 16
HD = 128
T = 2048
D = 2048
I = 5504
BT = 256
NT = T // BT
NJ = I // 128
SCALE = 1.0 / (HD ** 0.5)
NEG = -1e30
F32 = jnp.float32
BF16 = jnp.bfloat16


def _silu(x):
    return x / (1.0 + jnp.exp(-x))


def _addnorm1_body(h_ref, r_ref, w_ref, z_ref, xn_ref):
    z = h_ref[...] + r_ref[...]
    z_ref[...] = z
    v = jnp.mean(z * z, axis=1, keepdims=True)
    xn_ref[...] = (z * jax.lax.rsqrt(v + EPS) * w_ref[...]).astype(BF16)


def _addnorm1(h, r, ln1_row):
    return pl.pallas_call(
        _addnorm1_body,
        grid=(NT,),
        in_specs=[
            pl.BlockSpec((BT, D), lambda t: (t, 0)),
            pl.BlockSpec((BT, D), lambda t: (t, 0)),
            pl.BlockSpec((1, D), lambda t: (0, 0)),
        ],
        out_specs=[
            pl.BlockSpec((BT, D), lambda t: (t, 0)),
            pl.BlockSpec((BT, D), lambda t: (t, 0)),
        ],
        out_shape=[
            jax.ShapeDtypeStruct((T, D), F32),
            jax.ShapeDtypeStruct((T, D), BF16),
        ],
    )(h, r, ln1_row)


def _qkv_body(xn_ref, pos_ref, qw_ref, kw_ref, vw_ref, qb_ref, kb_ref, vb_ref,
              q_ref, k_ref, v_ref, cos_ref, sin_ref):
    j = pl.program_id(0)

    @pl.when(j == 0)
    def _():
        li = jax.lax.broadcasted_iota(jnp.int32, (T, HD), 1)
        f = jnp.where(li < HD // 2, li, li - HD // 2).astype(F32)
        inv = jnp.exp(f * (-jnp.log(THETA) / (HD // 2)))
        freqs = pos_ref[...] * inv
        cos_ref[...] = jnp.cos(freqs)
        sgn = jnp.where(li < HD // 2, -1.0, 1.0)
        sin_ref[...] = jnp.sin(freqs) * sgn

    xn = xn_ref[...]

    def proj(w_ref, b_ref):
        w = w_ref[...].astype(BF16)
        y = jnp.dot(xn, w, preferred_element_type=F32)
        return y + b_ref[0]

    def rope(x):
        xr = jnp.concatenate([x[:, HD // 2:], x[:, :HD // 2]], axis=1)
        return x * cos_ref[...] + xr * sin_ref[...]

    q_ref[...] = rope(proj(qw_ref, qb_ref)).astype(BF16)
    k_ref[...] = rope(proj(kw_ref, kb_ref)).astype(BF16)
    v_ref[...] = proj(vw_ref, vb_ref).astype(BF16)


def _qkv(xn, pos_col, q_w, k_w, v_w, qb3, kb3, vb3):
    wspec = pl.BlockSpec((D, HD), lambda j: (0, j))
    bspec = pl.BlockSpec((1, 1, HD), lambda j: (j, 0, 0))
    ospec = pl.BlockSpec((T, HD), lambda j: (0, j))
    return pl.pallas_call(
        _qkv_body,
        grid=(H,),
        in_specs=[
            pl.BlockSpec((T, D), lambda j: (0, 0)),
            pl.BlockSpec((T, 1), lambda j: (0, 0)),
            wspec, wspec, wspec, bspec, bspec, bspec,
        ],
        out_specs=[ospec, ospec, ospec],
        out_shape=[jax.ShapeDtypeStruct((T, D), BF16)] * 3,
        scratch_shapes=[
            pltpu.VMEM((T, HD), F32),
            pltpu.VMEM((T, HD), F32),
        ],
        compiler_params=pltpu.CompilerParams(vmem_limit_bytes=100 * 2**20),
    )(xn, pos_col, q_w, k_w, v_w, qb3, kb3, vb3)


def _attn_body(q_ref, k_ref, v_ref, o_ref):
    qt = pl.program_id(1)
    q = q_ref[...]
    s = jax.lax.dot_general(q, k_ref[...], (((1,), (1,)), ((), ())),
                            preferred_element_type=F32) * SCALE
    row = qt * BT + jax.lax.broadcasted_iota(jnp.int32, (BT, T), 0)
    col = jax.lax.broadcasted_iota(jnp.int32, (BT, T), 1)
    s = jnp.where(row >= col, s, NEG)
    m = jnp.max(s, axis=1, keepdims=True)
    p = jnp.exp(s - m)
    p = p / jnp.sum(p, axis=1, keepdims=True)
    o = jnp.dot(p.astype(BF16), v_ref[...], preferred_element_type=F32)
    o_ref[...] = o.astype(BF16)


def _attention(q, k, v):
    return pl.pallas_call(
        _attn_body,
        grid=(H, NT),
        in_specs=[
            pl.BlockSpec((BT, HD), lambda h, t: (t, h)),
            pl.BlockSpec((T, HD), lambda h, t: (0, h)),
            pl.BlockSpec((T, HD), lambda h, t: (0, h)),
        ],
        out_specs=pl.BlockSpec((BT, HD), lambda h, t: (t, h)),
        out_shape=jax.ShapeDtypeStruct((T, D), BF16),
    )(q, k, v)


def _onorm_body(a_ref, ow_ref, z_ref, w2_ref, mu_ref, mg_ref,
                z2_ref, xu_ref, xg_ref, owb_ref):
    t = pl.program_id(0)

    @pl.when(t == 0)
    def _():
        owb_ref[...] = ow_ref[...].astype(BF16)

    ao = jnp.dot(a_ref[...], owb_ref[...], preferred_element_type=F32)
    z2 = ao + z_ref[...]
    z2_ref[...] = z2
    v = jnp.mean(z2 * z2, axis=1, keepdims=True)
    h2 = z2 * jax.lax.rsqrt(v + EPS) * w2_ref[...]
    xu_ref[...] = (h2 * mu_ref[...]).astype(BF16)
    xg_ref[...] = (h2 * mg_ref[...]).astype(BF16)


def _onorm(attn, o_w, z, ln2_row, mu_col, mg_col):
    tspec = pl.BlockSpec((BT, D), lambda t: (t, 0))
    cspec = pl.BlockSpec((BT, 1), lambda t: (t, 0))
    return pl.pallas_call(
        _onorm_body,
        grid=(NT,),
        in_specs=[
            tspec,
            pl.BlockSpec((D, D), lambda t: (0, 0)),
            tspec,
            pl.BlockSpec((1, D), lambda t: (0, 0)),
            cspec, cspec,
        ],
        out_specs=[tspec, tspec, tspec],
        out_shape=[
            jax.ShapeDtypeStruct((T, D), F32),
            jax.ShapeDtypeStruct((T, D), BF16),
            jax.ShapeDtypeStruct((T, D), BF16),
        ],
        scratch_shapes=[pltpu.VMEM((D, D), BF16)],
        compiler_params=pltpu.CompilerParams(vmem_limit_bytes=100 * 2**20),
    )(attn, o_w, z, ln2_row, mu_col, mg_col)


def _moe_body(xu_ref, xg_ref, guw_ref, uuw_ref, duw_ref, ggw_ref, ugw_ref,
              dgw_ref, out_ref, wu_ref, wg_ref, du_ref, dg_ref):
    j = pl.program_id(0)
    wu_ref[:, :HD] = guw_ref[...].astype(BF16)
    wu_ref[:, HD:] = uuw_ref[...].astype(BF16)
    wg_ref[:, :HD] = ggw_ref[...].astype(BF16)
    wg_ref[:, HD:] = ugw_ref[...].astype(BF16)
    du_ref[...] = duw_ref[...].astype(BF16)
    dg_ref[...] = dgw_ref[...].astype(BF16)

    def expert(x_ref, w_ref, d_ref):
        gu = jnp.dot(x_ref[...], w_ref[...], preferred_element_type=F32)
        a = (_silu(gu[:, :HD]) * gu[:, HD:]).astype(BF16)
        return jnp.dot(a, d_ref[...], preferred_element_type=F32)

    o = expert(xu_ref, wu_ref, du_ref) + expert(xg_ref, wg_ref, dg_ref)

    @pl.when(j == 0)
    def _():
        out_ref[...] = o

    @pl.when(j > 0)
    def _():
        out_ref[...] += o


def _moe(xu, xg, gate_w, up_w, down_w, gen_gate_w, gen_up_w, gen_down_w):
    xspec = pl.BlockSpec((T, D), lambda j: (0, 0))
    gspec = pl.BlockSpec((D, HD), lambda j: (0, j))
    dspec = pl.BlockSpec((HD, D), lambda j: (j, 0))
    return pl.pallas_call(
        _moe_body,
        grid=(NJ,),
        in_specs=[xspec, xspec, gspec, gspec, dspec, gspec, gspec, dspec],
        out_specs=pl.BlockSpec((T, D), lambda j: (0, 0)),
        out_shape=jax.ShapeDtypeStruct((T, D), F32),
        scratch_shapes=[
            pltpu.VMEM((D, 2 * HD), BF16),
            pltpu.VMEM((D, 2 * HD), BF16),
            pltpu.VMEM((HD, D), BF16),
            pltpu.VMEM((HD, D), BF16),
        ],
        compiler_params=pltpu.CompilerParams(
            dimension_semantics=("arbitrary",),
            vmem_limit_bytes=100 * 2**20,
        ),
    )(xu, xg, gate_w, up_w, down_w, gen_gate_w, gen_up_w, gen_down_w)


def kernel(positions, hidden_states, residual, gen_token_mask, ln1_w, ln2_w,
           q_w, q_b, k_w, k_b, v_w, v_b, o_w, gate_w, up_w, down_w,
           gen_gate_w, gen_up_w, gen_down_w):
    pos_col = positions.astype(F32).reshape(T, 1)
    ln1_row = ln1_w.reshape(1, D)
    ln2_row = ln2_w.reshape(1, D)
    qb3 = q_b.reshape(H, 1, HD)
    kb3 = k_b.reshape(H, 1, HD)
    vb3 = v_b.reshape(H, 1, HD)
    mg_col = gen_token_mask.astype(F32).reshape(T, 1)
    mu_col = 1.0 - mg_col

    z, xn = _addnorm1(hidden_states, residual, ln1_row)
    q, k, v = _qkv(xn, pos_col, q_w, k_w, v_w, qb3, kb3, vb3)
    attn = _attention(q, k, v)
    z2, xu, xg = _onorm(attn, o_w, z, ln2_row, mu_col, mg_col)
    out = _moe(xu, xg, gate_w, up_w, down_w, gen_gate_w, gen_up_w, gen_down_w)
    return (out, z2)
